# baseline (speedup 1.0000x reference)
import functools

import jax
import jax.numpy as jnp
from jax import lax
from jax.experimental import pallas as pl
from jax.experimental.pallas import tpu as pltpu
from jax.experimental.pallas import tpu_sc as plsc

E = 16
TOPK = 2
H = 2048
FF = 1024
SF = 1024
T = 2048
A = T * TOPK
TM = 128
NT = (A + E * TM) // TM
NPAD = NT * TM

TMS = 256


def _silu(g):
    return g * (1.0 / (1.0 + jnp.exp(-g)))


def _moe_tile_kernel(eot_ref, cf_ref, rb_ref, xs_ref, w1_ref, w3_ref, w2_ref,
                     wt_ref, ys_ref):
    i = pl.program_id(0)

    @pl.when(cf_ref[i] > 0)
    def _():
        xb = xs_ref[...]
        g = jax.lax.dot_general(xb, w1_ref[0], (((1,), (1,)), ((), ())),
                                preferred_element_type=jnp.float32)
        u = jax.lax.dot_general(xb, w3_ref[0], (((1,), (1,)), ((), ())),
                                preferred_element_type=jnp.float32)
        h = _silu(g) * u
        y = jax.lax.dot_general(h, w2_ref[0], (((1,), (1,)), ((), ())),
                                preferred_element_type=jnp.float32)
        ys_ref[...] = y * wt_ref[0, 0, :][:, None]


def _prep_kernel(x_ref, gw_ref, b_ref, wg_ref, wu_ref, wd_ref,
                 sh_ref, meta_ref, cnt_ref):
    i = pl.program_id(0)
    xb = x_ref[...]

    logits = jax.lax.dot_general(xb, gw_ref[...], (((1,), (1,)), ((), ())),
                                 preferred_element_type=jnp.float32)
    scores = 1.0 / (1.0 + jnp.exp(-logits))
    biased = scores + b_ref[0:1, :]
    ei = jax.lax.broadcasted_iota(jnp.int32, (TMS, E), 1)
    m1 = jnp.max(biased, axis=1, keepdims=True)
    i1 = jnp.min(jnp.where(biased == m1, ei, E), axis=1, keepdims=True)
    oh1 = ei == i1
    masked = jnp.where(oh1, -jnp.inf, biased)
    m2 = jnp.max(masked, axis=1, keepdims=True)
    i2 = jnp.min(jnp.where(masked == m2, ei, E), axis=1, keepdims=True)
    oh2 = ei == i2
    s1 = jnp.sum(jnp.where(oh1, scores, 0.0), axis=1, keepdims=True)
    s2 = jnp.sum(jnp.where(oh2, scores, 0.0), axis=1, keepdims=True)
    ssum = s1 + s2
    wa = s1 / ssum
    wb = s2 / ssum

    inc = oh1.astype(jnp.float32) + oh2.astype(jnp.float32)
    r = jax.lax.broadcasted_iota(jnp.int32, (TMS, TMS), 0)
    c = jax.lax.broadcasted_iota(jnp.int32, (TMS, TMS), 1)
    tri = (c < r).astype(jnp.float32)
    excl_local = jax.lax.dot_general(tri, inc, (((1,), (0,)), ((), ())),
                                     preferred_element_type=jnp.float32)

    @pl.when(i == 0)
    def _():
        cnt_ref[...] = jnp.zeros_like(cnt_ref)

    base = cnt_ref[0:1, 0:E]
    excl = excl_local + base
    rank1 = jnp.sum(jnp.where(oh1, excl, 0.0), axis=1, keepdims=True)
    rank2 = jnp.sum(jnp.where(oh2, excl, 0.0), axis=1, keepdims=True)
    cnt_ref[0:1, 0:E] = base + jnp.sum(inc, axis=0, keepdims=True)

    col = jax.lax.broadcasted_iota(jnp.int32, (TMS, 128), 1)
    meta = jnp.where(col == 0, i1.astype(jnp.float32), 0.0)
    meta = jnp.where(col == 1, i2.astype(jnp.float32), meta)
    meta = jnp.where(col == 2, wa, meta)
    meta = jnp.where(col == 3, wb, meta)
    meta = jnp.where(col == 4, rank1, meta)
    meta = jnp.where(col == 5, rank2, meta)
    meta_ref[...] = meta

    g = jax.lax.dot_general(xb, wg_ref[...], (((1,), (1,)), ((), ())),
                            preferred_element_type=jnp.float32)
    u = jax.lax.dot_general(xb, wu_ref[...], (((1,), (1,)), ((), ())),
                            preferred_element_type=jnp.float32)
    h = _silu(g) * u
    sh_ref[...] = jax.lax.dot_general(h, wd_ref[...], (((1,), (1,)), ((), ())),
                                      preferred_element_type=jnp.float32)


NW = 32
TW = T // NW
CC = 8


def _combine_body(ys_hbm, sh_hbm, pos_hbm, out_hbm, idx_v, rows0, rows1,
                  sh0, sh1, out_v, semg0, semg1, sems0, sems1):
    wid = lax.axis_index("s") * 2 + lax.axis_index("c")
    tbase = wid * TW
    nch = TW // CC
    rows = (rows0, rows1)
    shb = (sh0, sh1)
    semg = (semg0, semg1)
    sems = (sems0, sems1)
    pltpu.sync_copy(pos_hbm.at[pl.ds(2 * tbase, 2 * TW)], idx_v)

    def fetch(ch):
        g = pltpu.async_copy(
            ys_hbm.at[idx_v.at[pl.ds(ch * 2 * CC, 2 * CC)]],
            rows[ch % 2], semg[ch % 2])
        s = pltpu.async_copy(sh_hbm.at[pl.ds(tbase + ch * CC, CC)],
                             shb[ch % 2], sems[ch % 2])
        return g, s

    cps = {0: fetch(0)}
    for ch in range(nch):
        if ch + 1 < nch:
            cps[ch + 1] = fetch(ch + 1)
        g, s = cps[ch]
        g.wait()
        s.wait()
        rbuf = rows[ch % 2]
        sbuf = shb[ch % 2]

        def body(j, _):
            sl = pl.ds(j * 16, 16)
            for r in range(CC):
                out_v[r, sl] = (rbuf[2 * r, sl] + rbuf[2 * r + 1, sl]
                                + sbuf[r, sl])
            return 0
        lax.fori_loop(0, H // 16, body, 0)
        pltpu.sync_copy(out_v, out_hbm.at[pl.ds(tbase + ch * CC, CC)])


SCC = 16


def _dispatch_body(x_hbm, d1_hbm, d2_hbm, xs_hbm, i1c, i2c, rows0, rows1,
                   seml0, seml1, sems):
    wid = lax.axis_index("s") * 2 + lax.axis_index("c")
    tbase = wid * TW
    nch = TW // SCC
    rows = (rows0, rows1)
    seml = (seml0, seml1)

    def load(ch):
        return pltpu.async_copy(x_hbm.at[pl.ds(tbase + ch * SCC, SCC)],
                                rows[ch % 2], seml[ch % 2])

    loads = {0: load(0)}
    for ch in range(nch):
        base = tbase + ch * SCC
        if ch + 1 < nch:
            loads[ch + 1] = load(ch + 1)
        pltpu.sync_copy(d1_hbm.at[pl.ds(base, SCC)], i1c)
        pltpu.sync_copy(d2_hbm.at[pl.ds(base, SCC)], i2c)
        loads[ch].wait()
        c1 = pltpu.async_copy(rows[ch % 2], xs_hbm.at[i1c], sems)
        c2 = pltpu.async_copy(rows[ch % 2], xs_hbm.at[i2c], sems)
        c1.wait()
        c2.wait()


def _sc_dispatch(x, dest1, dest2):
    mesh = plsc.VectorSubcoreMesh(core_axis_name="c", subcore_axis_name="s")
    return pl.kernel(
        _dispatch_body,
        mesh=mesh,
        out_type=jax.ShapeDtypeStruct((NPAD, H), jnp.float32),
        scratch_types=[
            pltpu.VMEM((SCC,), jnp.int32),
            pltpu.VMEM((SCC,), jnp.int32),
            pltpu.VMEM((SCC, H), jnp.float32),
            pltpu.VMEM((SCC, H), jnp.float32),
            pltpu.SemaphoreType.DMA,
            pltpu.SemaphoreType.DMA,
            pltpu.SemaphoreType.DMA,
        ],
    )(x, dest1, dest2)


def _sc_combine(ys, shared, posflat):
    mesh = plsc.VectorSubcoreMesh(core_axis_name="c", subcore_axis_name="s")
    return pl.kernel(
        _combine_body,
        mesh=mesh,
        out_type=jax.ShapeDtypeStruct((T, H), jnp.float32),
        scratch_types=[
            pltpu.VMEM((2 * TW,), jnp.int32),
            pltpu.VMEM((2 * CC, H), jnp.float32),
            pltpu.VMEM((2 * CC, H), jnp.float32),
            pltpu.VMEM((CC, H), jnp.float32),
            pltpu.VMEM((CC, H), jnp.float32),
            pltpu.VMEM((CC, H), jnp.float32),
            pltpu.SemaphoreType.DMA,
            pltpu.SemaphoreType.DMA,
            pltpu.SemaphoreType.DMA,
            pltpu.SemaphoreType.DMA,
        ],
    )(ys, shared, posflat)


def kernel(hidden_states, gate_w, bias, w1, w3, w2, sh_wg, sh_wu, sh_wd):
    x = hidden_states

    bias_b = jnp.broadcast_to(bias, (8, E))
    shared, meta, cnt = pl.pallas_call(
        _prep_kernel,
        grid=(T // TMS,),
        in_specs=[
            pl.BlockSpec((TMS, H), lambda i: (i, 0)),
            pl.BlockSpec((E, H), lambda i: (0, 0)),
            pl.BlockSpec((8, E), lambda i: (0, 0)),
            pl.BlockSpec((SF, H), lambda i: (0, 0)),
            pl.BlockSpec((SF, H), lambda i: (0, 0)),
            pl.BlockSpec((H, SF), lambda i: (0, 0)),
        ],
        out_specs=[
            pl.BlockSpec((TMS, H), lambda i: (i, 0)),
            pl.BlockSpec((TMS, 128), lambda i: (i, 0)),
            pl.BlockSpec((8, 128), lambda i: (0, 0)),
        ],
        out_shape=[
            jax.ShapeDtypeStruct((T, H), jnp.float32),
            jax.ShapeDtypeStruct((T, 128), jnp.float32),
            jax.ShapeDtypeStruct((8, 128), jnp.float32),
        ],
    )(x, gate_w, bias_b, sh_wg, sh_wu, sh_wd)

    e1 = meta[:, 0].astype(jnp.int32)
    e2 = meta[:, 1].astype(jnp.int32)
    topk_w = meta[:, 2:4]
    r1 = meta[:, 4].astype(jnp.int32)
    r2 = meta[:, 5].astype(jnp.int32)
    counts = cnt[0, :E].astype(jnp.int32)

    padded = ((counts + TM - 1) // TM) * TM
    offs = jnp.concatenate([jnp.zeros(1, jnp.int32),
                            jnp.cumsum(padded)[:-1].astype(jnp.int32)])
    dest1 = offs[e1] + r1
    dest2 = offs[e2] + r2
    pos = jnp.stack([dest1, dest2], axis=1)
    wts_p = (jnp.zeros(NPAD, jnp.float32).at[dest1].set(topk_w[:, 0])
             .at[dest2].set(topk_w[:, 1])).reshape(NT, 1, TM)

    tend = jnp.cumsum(padded // TM).astype(jnp.int32)
    tidx = jnp.arange(NT, dtype=jnp.int32)
    raw_e = jnp.sum((tend[None, :] <= tidx[:, None]).astype(jnp.int32), axis=1)
    last_tile = tend[-1] - 1
    last_e = jnp.sum((tend <= last_tile).astype(jnp.int32))
    valid = tidx < tend[-1]
    tile_expert = jnp.where(valid, raw_e, last_e).astype(jnp.int32)
    tile_valid = valid.astype(jnp.int32)
    row_block = jnp.where(valid, tidx, last_tile).astype(jnp.int32)

    xs = _sc_dispatch(x, dest1, dest2)

    ys = pl.pallas_call(
        _moe_tile_kernel,
        grid_spec=pltpu.PrefetchScalarGridSpec(
            num_scalar_prefetch=3,
            grid=(NT,),
            in_specs=[
                pl.BlockSpec((TM, H), lambda i, eot, cf, rb: (rb[i], 0)),
                pl.BlockSpec((1, FF, H),
                             lambda i, eot, cf, rb: (eot[i], 0, 0)),
                pl.BlockSpec((1, FF, H),
                             lambda i, eot, cf, rb: (eot[i], 0, 0)),
                pl.BlockSpec((1, H, FF),
                             lambda i, eot, cf, rb: (eot[i], 0, 0)),
                pl.BlockSpec((1, 1, TM),
                             lambda i, eot, cf, rb: (rb[i], 0, 0)),
            ],
            out_specs=pl.BlockSpec((TM, H), lambda i, eot, cf, rb: (rb[i], 0)),
        ),
        out_shape=jax.ShapeDtypeStruct((NPAD, H), jnp.float32),
    )(tile_expert, tile_valid, row_block, xs, w1, w3, w2, wts_p)

    return _sc_combine(ys, shared, pos.reshape(-1))


# speedup vs baseline: 1.2630x; 1.2630x over previous
import functools

import jax
import jax.numpy as jnp
from jax import lax
from jax.experimental import pallas as pl
from jax.experimental.pallas import tpu as pltpu
from jax.experimental.pallas import tpu_sc as plsc

E = 16
TOPK = 2
H = 2048
FF = 1024
SF = 1024
T = 2048
A = T * TOPK
TM = 256
NT = (A + E * TM) // TM
NPAD = NT * TM

TMS = 256


def _silu(g):
    return g * (1.0 / (1.0 + jnp.exp(-g)))


def _moe_tile_kernel(eot_ref, cf_ref, rb_ref, xs_ref, w1_ref, w3_ref, w2_ref,
                     ys_ref):
    i = pl.program_id(0)

    @pl.when(cf_ref[i] > 0)
    def _():
        xb = xs_ref[...]
        g = jax.lax.dot_general(xb, w1_ref[0], (((1,), (1,)), ((), ())),
                                preferred_element_type=jnp.float32)
        u = jax.lax.dot_general(xb, w3_ref[0], (((1,), (1,)), ((), ())),
                                preferred_element_type=jnp.float32)
        h = _silu(g) * u
        ys_ref[...] = jax.lax.dot_general(h, w2_ref[0], (((1,), (1,)), ((), ())),
                                          preferred_element_type=jnp.float32)


def _prep_kernel(x_ref, gw_ref, b_ref, wg_ref, wu_ref, wd_ref,
                 sh_ref, meta_ref, cnt_ref):
    i = pl.program_id(0)
    xb = x_ref[...]

    logits = jax.lax.dot_general(xb, gw_ref[...], (((1,), (1,)), ((), ())),
                                 preferred_element_type=jnp.float32)
    scores = 1.0 / (1.0 + jnp.exp(-logits))
    biased = scores + b_ref[0:1, :]
    ei = jax.lax.broadcasted_iota(jnp.int32, (TMS, E), 1)
    m1 = jnp.max(biased, axis=1, keepdims=True)
    i1 = jnp.min(jnp.where(biased == m1, ei, E), axis=1, keepdims=True)
    oh1 = ei == i1
    masked = jnp.where(oh1, -jnp.inf, biased)
    m2 = jnp.max(masked, axis=1, keepdims=True)
    i2 = jnp.min(jnp.where(masked == m2, ei, E), axis=1, keepdims=True)
    oh2 = ei == i2
    s1 = jnp.sum(jnp.where(oh1, scores, 0.0), axis=1, keepdims=True)
    s2 = jnp.sum(jnp.where(oh2, scores, 0.0), axis=1, keepdims=True)
    ssum = s1 + s2
    wa = s1 / ssum
    wb = s2 / ssum

    inc = oh1.astype(jnp.float32) + oh2.astype(jnp.float32)
    r = jax.lax.broadcasted_iota(jnp.int32, (TMS, TMS), 0)
    c = jax.lax.broadcasted_iota(jnp.int32, (TMS, TMS), 1)
    tri = (c < r).astype(jnp.float32)
    excl_local = jax.lax.dot_general(tri, inc, (((1,), (0,)), ((), ())),
                                     preferred_element_type=jnp.float32)

    @pl.when(i == 0)
    def _():
        cnt_ref[...] = jnp.zeros_like(cnt_ref)

    base = cnt_ref[0:1, 0:E]
    excl = excl_local + base
    rank1 = jnp.sum(jnp.where(oh1, excl, 0.0), axis=1, keepdims=True)
    rank2 = jnp.sum(jnp.where(oh2, excl, 0.0), axis=1, keepdims=True)
    cnt_ref[0:1, 0:E] = base + jnp.sum(inc, axis=0, keepdims=True)

    col = jax.lax.broadcasted_iota(jnp.int32, (TMS, 128), 1)
    meta = jnp.where(col == 0, i1.astype(jnp.float32), 0.0)
    meta = jnp.where(col == 1, i2.astype(jnp.float32), meta)
    meta = jnp.where(col == 2, wa, meta)
    meta = jnp.where(col == 3, wb, meta)
    meta = jnp.where(col == 4, rank1, meta)
    meta = jnp.where(col == 5, rank2, meta)
    meta_ref[...] = meta

    g = jax.lax.dot_general(xb, wg_ref[...], (((1,), (1,)), ((), ())),
                            preferred_element_type=jnp.float32)
    u = jax.lax.dot_general(xb, wu_ref[...], (((1,), (1,)), ((), ())),
                            preferred_element_type=jnp.float32)
    h = _silu(g) * u
    sh_ref[...] = jax.lax.dot_general(h, wd_ref[...], (((1,), (1,)), ((), ())),
                                      preferred_element_type=jnp.float32)


NW = 32
TW = T // NW
CC = 8


def _combine_body(ys_hbm, sh_hbm, pos_hbm, w_hbm, out_hbm, idx_v, wv_v,
                  rows0, rows1, sh0, sh1, out_v, semg0, semg1, sems0, sems1):
    wid = lax.axis_index("s") * 2 + lax.axis_index("c")
    tbase = wid * TW
    nch = TW // CC
    rows = (rows0, rows1)
    shb = (sh0, sh1)
    semg = (semg0, semg1)
    sems = (sems0, sems1)
    pltpu.sync_copy(pos_hbm.at[pl.ds(2 * tbase, 2 * TW)], idx_v)
    pltpu.sync_copy(w_hbm.at[pl.ds(2 * tbase, 2 * TW)], wv_v)

    def fetch(ch):
        g = pltpu.async_copy(
            ys_hbm.at[idx_v.at[pl.ds(ch * 2 * CC, 2 * CC)]],
            rows[ch % 2], semg[ch % 2])
        s = pltpu.async_copy(sh_hbm.at[pl.ds(tbase + ch * CC, CC)],
                             shb[ch % 2], sems[ch % 2])
        return g, s

    cps = {0: fetch(0)}
    for ch in range(nch):
        if ch + 1 < nch:
            cps[ch + 1] = fetch(ch + 1)
        g, s = cps[ch]
        g.wait()
        s.wait()
        rbuf = rows[ch % 2]
        sbuf = shb[ch % 2]

        wvec = wv_v[pl.ds(ch * 2 * CC, 2 * CC)]

        def body(j, _):
            sl = pl.ds(j * 16, 16)
            for r in range(CC):
                out_v[r, sl] = (wvec[2 * r] * rbuf[2 * r, sl]
                                + wvec[2 * r + 1] * rbuf[2 * r + 1, sl]
                                + sbuf[r, sl])
            return 0
        lax.fori_loop(0, H // 16, body, 0)
        pltpu.sync_copy(out_v, out_hbm.at[pl.ds(tbase + ch * CC, CC)])


SCC = 16


def _dispatch_body(x_hbm, d1_hbm, d2_hbm, xs_hbm, i1c, i2c, rows0, rows1,
                   seml0, seml1, sems):
    wid = lax.axis_index("s") * 2 + lax.axis_index("c")
    tbase = wid * TW
    nch = TW // SCC
    rows = (rows0, rows1)
    seml = (seml0, seml1)

    def load(ch):
        return pltpu.async_copy(x_hbm.at[pl.ds(tbase + ch * SCC, SCC)],
                                rows[ch % 2], seml[ch % 2])

    loads = {0: load(0)}
    for ch in range(nch):
        base = tbase + ch * SCC
        if ch + 1 < nch:
            loads[ch + 1] = load(ch + 1)
        pltpu.sync_copy(d1_hbm.at[pl.ds(base, SCC)], i1c)
        pltpu.sync_copy(d2_hbm.at[pl.ds(base, SCC)], i2c)
        loads[ch].wait()
        c1 = pltpu.async_copy(rows[ch % 2], xs_hbm.at[i1c], sems)
        c2 = pltpu.async_copy(rows[ch % 2], xs_hbm.at[i2c], sems)
        c1.wait()
        c2.wait()


def _sc_dispatch(x, dest1, dest2):
    mesh = plsc.VectorSubcoreMesh(core_axis_name="c", subcore_axis_name="s")
    return pl.kernel(
        _dispatch_body,
        mesh=mesh,
        out_type=jax.ShapeDtypeStruct((NPAD, H), jnp.float32),
        scratch_types=[
            pltpu.VMEM((SCC,), jnp.int32),
            pltpu.VMEM((SCC,), jnp.int32),
            pltpu.VMEM((SCC, H), jnp.float32),
            pltpu.VMEM((SCC, H), jnp.float32),
            pltpu.SemaphoreType.DMA,
            pltpu.SemaphoreType.DMA,
            pltpu.SemaphoreType.DMA,
        ],
    )(x, dest1, dest2)


def _sc_combine(ys, shared, posflat, wflat):
    mesh = plsc.VectorSubcoreMesh(core_axis_name="c", subcore_axis_name="s")
    return pl.kernel(
        _combine_body,
        mesh=mesh,
        out_type=jax.ShapeDtypeStruct((T, H), jnp.float32),
        scratch_types=[
            pltpu.VMEM((2 * TW,), jnp.int32),
            pltpu.VMEM((2 * TW,), jnp.float32),
            pltpu.VMEM((2 * CC, H), jnp.float32),
            pltpu.VMEM((2 * CC, H), jnp.float32),
            pltpu.VMEM((CC, H), jnp.float32),
            pltpu.VMEM((CC, H), jnp.float32),
            pltpu.VMEM((CC, H), jnp.float32),
            pltpu.SemaphoreType.DMA,
            pltpu.SemaphoreType.DMA,
            pltpu.SemaphoreType.DMA,
            pltpu.SemaphoreType.DMA,
        ],
    )(ys, shared, posflat, wflat)


def kernel(hidden_states, gate_w, bias, w1, w3, w2, sh_wg, sh_wu, sh_wd):
    x = hidden_states

    bias_b = jnp.broadcast_to(bias, (8, E))
    shared, meta, cnt = pl.pallas_call(
        _prep_kernel,
        grid=(T // TMS,),
        in_specs=[
            pl.BlockSpec((TMS, H), lambda i: (i, 0)),
            pl.BlockSpec((E, H), lambda i: (0, 0)),
            pl.BlockSpec((8, E), lambda i: (0, 0)),
            pl.BlockSpec((SF, H), lambda i: (0, 0)),
            pl.BlockSpec((SF, H), lambda i: (0, 0)),
            pl.BlockSpec((H, SF), lambda i: (0, 0)),
        ],
        out_specs=[
            pl.BlockSpec((TMS, H), lambda i: (i, 0)),
            pl.BlockSpec((TMS, 128), lambda i: (i, 0)),
            pl.BlockSpec((8, 128), lambda i: (0, 0)),
        ],
        out_shape=[
            jax.ShapeDtypeStruct((T, H), jnp.float32),
            jax.ShapeDtypeStruct((T, 128), jnp.float32),
            jax.ShapeDtypeStruct((8, 128), jnp.float32),
        ],
    )(x, gate_w, bias_b, sh_wg, sh_wu, sh_wd)

    e1 = meta[:, 0].astype(jnp.int32)
    e2 = meta[:, 1].astype(jnp.int32)
    topk_w = meta[:, 2:4]
    r1 = meta[:, 4].astype(jnp.int32)
    r2 = meta[:, 5].astype(jnp.int32)
    counts = cnt[0, :E].astype(jnp.int32)

    padded = ((counts + TM - 1) // TM) * TM
    offs = jnp.concatenate([jnp.zeros(1, jnp.int32),
                            jnp.cumsum(padded)[:-1].astype(jnp.int32)])
    dest1 = offs[e1] + r1
    dest2 = offs[e2] + r2
    pos = jnp.stack([dest1, dest2], axis=1)

    tend = jnp.cumsum(padded // TM).astype(jnp.int32)
    tidx = jnp.arange(NT, dtype=jnp.int32)
    raw_e = jnp.sum((tend[None, :] <= tidx[:, None]).astype(jnp.int32), axis=1)
    last_tile = tend[-1] - 1
    last_e = jnp.sum((tend <= last_tile).astype(jnp.int32))
    valid = tidx < tend[-1]
    tile_expert = jnp.where(valid, raw_e, last_e).astype(jnp.int32)
    tile_valid = valid.astype(jnp.int32)
    row_block = jnp.where(valid, tidx, last_tile).astype(jnp.int32)

    xs = _sc_dispatch(x, dest1, dest2)

    ys = pl.pallas_call(
        _moe_tile_kernel,
        grid_spec=pltpu.PrefetchScalarGridSpec(
            num_scalar_prefetch=3,
            grid=(NT,),
            in_specs=[
                pl.BlockSpec((TM, H), lambda i, eot, cf, rb: (rb[i], 0)),
                pl.BlockSpec((1, FF, H),
                             lambda i, eot, cf, rb: (eot[i], 0, 0)),
                pl.BlockSpec((1, FF, H),
                             lambda i, eot, cf, rb: (eot[i], 0, 0)),
                pl.BlockSpec((1, H, FF),
                             lambda i, eot, cf, rb: (eot[i], 0, 0)),
            ],
            out_specs=pl.BlockSpec((TM, H), lambda i, eot, cf, rb: (rb[i], 0)),
        ),
        out_shape=jax.ShapeDtypeStruct((NPAD, H), jnp.float32),
    )(tile_expert, tile_valid, row_block, xs, w1, w3, w2)

    return _sc_combine(ys, shared, pos.reshape(-1), topk_w.reshape(-1))
